# baseline (speedup 1.0000x reference)
import functools

import jax
import jax.numpy as jnp
from jax import lax
from jax.experimental import pallas as pl
from jax.experimental.pallas import tpu as pltpu
from jax.experimental.pallas import tpu_sc as plsc

_FRACS = (0.25, 0.5, 0.75, 0.9)
_LN = 16


def _sc_body(attn_hbm, thr_hbm, o0, o1, o2, o3, vals_v, thr_v, orow_v, *,
             row, col_al, win_off, tt, bpw):
    wid = lax.axis_index("s") * 2 + lax.axis_index("c")
    pltpu.sync_copy(thr_hbm, thr_v)
    thrs = thr_v[...]
    outs = (o0, o1, o2, o3)

    def body(t, carry):
        b = wid * bpw + t
        pltpu.sync_copy(attn_hbm.at[b, row, pl.ds(col_al, 56)],
                        vals_v.at[pl.ds(0, 56)])
        vcs = [vals_v[pl.ds(win_off + _LN * c, _LN)] for c in range(4)]
        gidx = [lax.iota(jnp.int32, _LN) + _LN * c for c in range(4)]
        ranks = [jnp.zeros((_LN,), jnp.int32) for _ in range(4)]
        one = jnp.ones((_LN,), jnp.int32)
        zero = jnp.zeros((_LN,), jnp.int32)
        for jc in range(4):
            for l in range(_LN):
                j = _LN * jc + l
                if j >= tt:
                    break
                sv = jnp.full((_LN,), vcs[jc][l], jnp.float32)
                for c in range(4):
                    cmp = (sv < vcs[c]) | ((sv == vcs[c]) & (gidx[c] > j))
                    ranks[c] = ranks[c] + jnp.where(cmp, one, zero)
        for k in range(4):
            thr = thrs[k]
            for c in range(4):
                m = ranks[c] >= thr
                orow_v[pl.ds(_LN * c, _LN)] = jnp.where(m, one, zero)
            pltpu.sync_copy(orow_v, outs[k].at[b])
        return carry

    lax.fori_loop(0, bpw, body, 0)


def kernel(attn_avg, inference, template_token_num):
    B, S, L = attn_avg.shape
    T = L - 200
    tt = 49
    row = (S - 200) + (14 // 2) * 14 + 14 // 2
    col0 = T - tt
    col_al = (col0 // 8) * 8
    win_off = col0 - col_al
    nw = 32
    bpw = B // nw

    zo = jnp.asarray(template_token_num, jnp.int32) - T
    thr = jnp.array([int(tt * f) for f in _FRACS] + [0] * 12, jnp.int32) + zo

    mesh = plsc.VectorSubcoreMesh(core_axis_name="c", subcore_axis_name="s")
    sck = functools.partial(
        pl.kernel,
        mesh=mesh,
        out_type=[jax.ShapeDtypeStruct((B, 64), jnp.int32)] * 4,
        scratch_types=[
            pltpu.VMEM((72,), jnp.float32),
            pltpu.VMEM((16,), jnp.int32),
            pltpu.VMEM((64,), jnp.int32),
        ],
    )(functools.partial(_sc_body, row=row, col_al=col_al, win_off=win_off,
                        tt=tt, bpw=bpw))
    outs = sck(attn_avg, thr)
    return tuple(o[:, :tt].astype(jnp.bool_) for o in outs)


# speedup vs baseline: 1.0087x; 1.0087x over previous
import functools

import jax
import jax.numpy as jnp
import numpy as np
from jax import lax
from jax.experimental import pallas as pl
from jax.experimental.pallas import tpu as pltpu
from jax.experimental.pallas import tpu_sc as plsc

_FRACS = (0.25, 0.5, 0.75, 0.9)
_LN = 16


def _sc_body(attn_hbm, thr_hbm, out_hbm, vals_v, thr_v, obuf_v, sem, *,
             row, col_al, win_off, tt, bpw):
    wid = lax.axis_index("s") * 2 + lax.axis_index("c")
    pltpu.sync_copy(thr_hbm, thr_v)
    thrs = thr_v[...]
    base = wid * bpw

    copies = [
        pltpu.make_async_copy(
            attn_hbm.at[base + t, row, pl.ds(col_al, 56)],
            vals_v.at[t, pl.ds(0, 56)], sem)
        for t in range(bpw)
    ]
    for c in copies:
        c.start()
    for c in copies:
        c.wait()

    one = jnp.ones((_LN,), jnp.int32)
    zero = jnp.zeros((_LN,), jnp.int32)
    for t in range(bpw):
        vcs = [vals_v[t, pl.ds(win_off + _LN * c, _LN)] for c in range(4)]
        gidx = [lax.iota(jnp.int32, _LN) + _LN * c for c in range(4)]
        ranks = [zero for _ in range(4)]
        for jc in range(4):
            for l in range(_LN):
                j = _LN * jc + l
                if j >= tt:
                    break
                sv = jnp.full((_LN,), vcs[jc][l], jnp.float32)
                for c in range(4):
                    cmp = (sv < vcs[c]) | ((sv == vcs[c]) & (gidx[c] > j))
                    ranks[c] = ranks[c] + jnp.where(cmp, one, zero)
        for k in range(4):
            thr = thrs[k]
            for c in range(4):
                m = ranks[c] >= thr
                obuf_v[t, k, pl.ds(_LN * c, _LN)] = jnp.where(m, one, zero)
    pltpu.sync_copy(obuf_v, out_hbm.at[pl.ds(base, bpw)])


def kernel(attn_avg, inference, template_token_num):
    B, S, L = attn_avg.shape
    T = L - 200
    tt = 49
    row = (S - 200) + (14 // 2) * 14 + 14 // 2
    col0 = T - tt
    col_al = (col0 // 8) * 8
    win_off = col0 - col_al
    nw = 32
    bpw = B // nw

    zo = jnp.asarray(template_token_num, jnp.int32) - T
    thr = jnp.array([int(tt * f) for f in _FRACS] + [0] * 12, jnp.int32) + zo

    mesh = plsc.VectorSubcoreMesh(core_axis_name="c", subcore_axis_name="s")
    sck = functools.partial(
        pl.kernel,
        mesh=mesh,
        out_type=jax.ShapeDtypeStruct((B, 4, 64), jnp.int32),
        scratch_types=[
            pltpu.VMEM((bpw, 72), jnp.float32),
            pltpu.VMEM((16,), jnp.int32),
            pltpu.VMEM((bpw, 4, 64), jnp.int32),
            pltpu.SemaphoreType.DMA,
        ],
    )(functools.partial(_sc_body, row=row, col_al=col_al, win_off=win_off,
                        tt=tt, bpw=bpw))
    out = sck(attn_avg, thr)
    return tuple(out[:, k, :tt].astype(jnp.bool_) for k in range(4))


# speedup vs baseline: 1.1686x; 1.1585x over previous
import jax
import jax.numpy as jnp
from jax import lax
from jax.experimental import pallas as pl
from jax.experimental.pallas import tpu as pltpu

_FRACS = (0.25, 0.5, 0.75, 0.9)


def _mask_kernel(zo_ref, x_ref, o0, o1, o2, o3, vs_ref, *, row_off, col_off,
                 tt):
    vs_ref[...] = x_ref[:, row_off, :]
    v = vs_ref[...]
    ones = jnp.ones(v.shape, jnp.int32)
    zeros = jnp.zeros(v.shape, jnp.int32)
    rank = zeros
    iota = lax.broadcasted_iota(jnp.int32, v.shape, 1)
    for j in range(col_off, col_off + tt):
        vj = v[:, j:j + 1]
        cmp = (vj < v) | ((vj == v) & (iota > j))
        rank = rank + jnp.where(cmp, ones, zeros)
    rank_w = rank[:, col_off:col_off + tt]
    zo = zo_ref[0, 0]
    for out, frac in zip((o0, o1, o2, o3), _FRACS):
        out[...] = rank_w >= int(tt * frac) + zo


def kernel(attn_avg, inference, template_token_num):
    B, S, L = attn_avg.shape
    T = L - 200
    tt = 49
    row = (S - 200) + (14 // 2) * 14 + 14 // 2
    col0 = T - tt
    r_blk = row // 8
    c_blk = col0 // 128
    assert col0 - c_blk * 128 + tt <= 128
    zero_offset = jnp.reshape(
        jnp.asarray(template_token_num, jnp.int32) - T, (1, 1))

    bb = B // 2
    out_sd = jax.ShapeDtypeStruct((B, tt), jnp.bool_)
    outs = pl.pallas_call(
        lambda zo, x, o0, o1, o2, o3, vs: _mask_kernel(
            zo, x, o0, o1, o2, o3, vs,
            row_off=row - r_blk * 8, col_off=col0 - c_blk * 128, tt=tt),
        grid=(2,),
        scratch_shapes=[pltpu.VMEM((bb, 128), jnp.float32)],
        in_specs=[
            pl.BlockSpec(memory_space=pltpu.SMEM),
            pl.BlockSpec((bb, 8, 128), lambda i: (i, r_blk, c_blk)),
        ],
        out_specs=[pl.BlockSpec((bb, tt), lambda i: (i, 0))] * 4,
        out_shape=[out_sd] * 4,
    )(zero_offset, attn_avg)
    return tuple(outs)
